# Initial kernel scaffold; baseline (speedup 1.0000x reference)
import jax
import jax.numpy as jnp
from jax.experimental import pallas as pl


def kernel(x, edge_index, Ws0, Wn0, b0, W1, b1, W2, b2, Ws3, Wn3, b3):
    raise NotImplementedError("write your pallas kernel here")


# speedup vs baseline: 7.6081x; 7.6081x over previous
import functools

import jax
import jax.numpy as jnp
from jax import lax
from jax.experimental import pallas as pl
from jax.experimental.pallas import tpu as pltpu
from jax.experimental.pallas import tpu_sc as plsc

N = 10000
D = 128
E = 320000
NC = 2
NS = 16
NW = NC * NS
K = 128
EPT = 10240
C = EPT // K
PAD_T = EPT - E // NW
NDUMMY = 112
NPAD = N + NDUMMY
ZPT = NPAD // NS
OPT = 1000
OTILES = N // OPT

_mesh = plsc.VectorSubcoreMesh(core_axis_name="c", subcore_axis_name="s")


def _agg_body(h_hbm, src_hbm, dst_hbm, outp, agg_sh, src_v, dst_v, rows_v,
              gsem):
    cid = lax.axis_index("c")
    sid = lax.axis_index("s")
    wid = cid * NS + sid

    zero16 = jnp.zeros((16,), jnp.float32)

    def _zrow(i, _):
        def _zcol(k, __):
            rows_v[i, pl.ds(k * 16, 16)] = zero16
            return __
        return lax.fori_loop(0, D // 16, _zcol, 0)
    lax.fori_loop(0, K, _zrow, 0)

    zb = sid * ZPT
    for t in range(ZPT // K):
        pltpu.sync_copy(rows_v, agg_sh.at[pl.ds(zb + t * K, K)])
    rem = ZPT % K
    if rem:
        pltpu.sync_copy(rows_v.at[pl.ds(0, rem)],
                        agg_sh.at[pl.ds(zb + (ZPT // K) * K, rem)])

    pltpu.sync_copy(src_hbm.at[wid], src_v)
    pltpu.sync_copy(dst_hbm.at[wid], dst_v)

    plsc.subcore_barrier()

    def _chunk(j, carry):
        pltpu.async_copy(h_hbm.at[src_v.at[j]], rows_v, gsem).wait()
        pltpu.sync_copy(rows_v, agg_sh.at[dst_v.at[j]], add=True)
        return carry
    lax.fori_loop(0, C, _chunk, 0)

    plsc.subcore_barrier()

    @pl.when(sid < OTILES)
    def _():
        ob = sid * OPT
        pltpu.sync_copy(agg_sh.at[pl.ds(ob, OPT)],
                        outp.at[cid, pl.ds(ob, OPT)])


_agg = pl.kernel(
    _agg_body,
    out_type=[jax.ShapeDtypeStruct((NC, N, D), jnp.float32)],
    mesh=_mesh,
    scratch_types=[
        pltpu.VMEM_SHARED((NPAD, D), jnp.float32),
        pltpu.VMEM((C, K), jnp.int32),
        pltpu.VMEM((C, K), jnp.int32),
        pltpu.VMEM((K, D), jnp.float32),
        pltpu.SemaphoreType.DMA,
    ])


def _deg_body(dst_hbm, degp, deg_sh, dst_v, rows_v):
    cid = lax.axis_index("c")
    sid = lax.axis_index("s")
    wid = cid * NS + sid

    zero16 = jnp.zeros((16,), jnp.float32)
    one16 = jnp.ones((16,), jnp.float32)

    def _fill(val):
        def _row(i, _):
            def _col(k, __):
                rows_v[i, pl.ds(k * 16, 16)] = val
                return __
            return lax.fori_loop(0, D // 16, _col, 0)
        lax.fori_loop(0, K, _row, 0)

    _fill(zero16)
    zb = sid * ZPT
    for t in range(ZPT // K):
        pltpu.sync_copy(rows_v, deg_sh.at[pl.ds(zb + t * K, K)])
    rem = ZPT % K
    if rem:
        pltpu.sync_copy(rows_v.at[pl.ds(0, rem)],
                        deg_sh.at[pl.ds(zb + (ZPT // K) * K, rem)])
    _fill(one16)

    pltpu.sync_copy(dst_hbm.at[wid], dst_v)

    plsc.subcore_barrier()

    def _chunk(j, carry):
        pltpu.sync_copy(rows_v, deg_sh.at[dst_v.at[j]], add=True)
        return carry
    lax.fori_loop(0, C, _chunk, 0)

    plsc.subcore_barrier()

    @pl.when(sid < OTILES)
    def _():
        ob = sid * OPT
        pltpu.sync_copy(deg_sh.at[pl.ds(ob, OPT)],
                        degp.at[cid, pl.ds(ob, OPT)])


_deg = pl.kernel(
    _deg_body,
    out_type=[jax.ShapeDtypeStruct((NC, N, D), jnp.float32)],
    mesh=_mesh,
    scratch_types=[
        pltpu.VMEM_SHARED((NPAD, D), jnp.float32),
        pltpu.VMEM((C, K), jnp.int32),
        pltpu.VMEM((K, D), jnp.float32),
    ])


BN = 1000
GRID = N // BN


def _norm_body(x_ref, o_ref):
    x = x_ref[...]
    nrm = jnp.sqrt(jnp.sum(x * x, axis=1, keepdims=True))
    o_ref[...] = x / jnp.maximum(nrm, 1e-12)


def _mean_body(h_ref, p0_ref, p1_ref, d0_ref, d1_ref, ws_ref, wn_ref, b_ref,
               o_ref, *, relu):
    deg = d0_ref[:, 0:1] + d1_ref[:, 0:1]
    hn = (p0_ref[...] + p1_ref[...]) / jnp.maximum(deg, 1.0)
    acc = jnp.dot(h_ref[...], ws_ref[...], preferred_element_type=jnp.float32)
    acc = acc + jnp.dot(hn, wn_ref[...], preferred_element_type=jnp.float32)
    acc = acc + b_ref[...]
    o_ref[...] = jnp.maximum(acc, 0.0) if relu else acc


def _gcn_body(h_ref, p0_ref, p1_ref, d0_ref, d1_ref, w_ref, b_ref, o_ref):
    deg = d0_ref[:, 0:1] + d1_ref[:, 0:1]
    h = h_ref[...]
    rst = (h + p0_ref[...] + p1_ref[...]) / (deg + 1.0)
    acc = jnp.dot(rst, w_ref[...], preferred_element_type=jnp.float32)
    o_ref[...] = jnp.maximum(acc + b_ref[...], 0.0)


def _row_spec(d):
    return pl.BlockSpec((BN, d), lambda i: (i, 0))


def _const_spec(r, c):
    return pl.BlockSpec((r, c), lambda i: (0, 0))


_norm_call = pl.pallas_call(
    _norm_body, grid=(GRID,),
    in_specs=[_row_spec(D)], out_specs=_row_spec(D),
    out_shape=jax.ShapeDtypeStruct((N, D), jnp.float32))


def _mean_call(dout, relu):
    return pl.pallas_call(
        functools.partial(_mean_body, relu=relu), grid=(GRID,),
        in_specs=[_row_spec(D), _row_spec(D), _row_spec(D),
                  _row_spec(D), _row_spec(D),
                  _const_spec(D, dout), _const_spec(D, dout),
                  _const_spec(1, dout)],
        out_specs=_row_spec(dout),
        out_shape=jax.ShapeDtypeStruct((N, dout), jnp.float32))


_mean0 = _mean_call(D, True)
_mean3 = _mean_call(64, False)

_gcn = pl.pallas_call(
    _gcn_body, grid=(GRID,),
    in_specs=[_row_spec(D), _row_spec(D), _row_spec(D),
              _row_spec(D), _row_spec(D),
              _const_spec(D, D), _const_spec(1, D)],
    out_specs=_row_spec(D),
    out_shape=jax.ShapeDtypeStruct((N, D), jnp.float32))


def kernel(x, edge_index, Ws0, Wn0, b0, W1, b1, W2, b2, Ws3, Wn3, b3):
    src = edge_index[0].astype(jnp.int32)
    dst = edge_index[1].astype(jnp.int32)
    pad = NW * PAD_T
    pad_src = (jnp.arange(pad, dtype=jnp.int32) * 131) % N
    pad_dst = N + jnp.arange(pad, dtype=jnp.int32) % NDUMMY
    src3 = jnp.concatenate(
        [src.reshape(NW, E // NW), pad_src.reshape(NW, PAD_T)],
        axis=1).reshape(NW, C, K)
    dst3 = jnp.concatenate(
        [dst.reshape(NW, E // NW), pad_dst.reshape(NW, PAD_T)],
        axis=1).reshape(NW, C, K)

    b0r = b0.reshape(1, D)
    b1r = b1.reshape(1, D)
    b2r = b2.reshape(1, D)
    b3r = b3.reshape(1, 64)

    dp, = _deg(dst3)
    d0, d1 = dp[0], dp[1]
    h = _norm_call(x)
    p, = _agg(h, src3, dst3)
    h = _mean0(h, p[0], p[1], d0, d1, Ws0.T, Wn0.T, b0r)
    p, = _agg(h, src3, dst3)
    h = _gcn(h, p[0], p[1], d0, d1, W1.T, b1r)
    p, = _agg(h, src3, dst3)
    h = _gcn(h, p[0], p[1], d0, d1, W2.T, b2r)
    p, = _agg(h, src3, dst3)
    out = _mean3(h, p[0], p[1], d0, d1, Ws3.T, Wn3.T, b3r)
    return out


# speedup vs baseline: 8.8541x; 1.1638x over previous
import functools

import jax
import jax.numpy as jnp
from jax import lax
from jax.experimental import pallas as pl
from jax.experimental.pallas import tpu as pltpu
from jax.experimental.pallas import tpu_sc as plsc

N = 10000
D = 128
E = 320000
NC = 2
NS = 16
NW = NC * NS
K = 128
EPT = 10240
C = EPT // K
G = 8
KD = 128
CD = EPT // KD
PAD_T = EPT - E // NW
NDUMMY = 16
NPAD = N + NDUMMY
ZPT = NPAD // NS
OPT = 1000
OTILES = N // OPT

_mesh = plsc.VectorSubcoreMesh(core_axis_name="c", subcore_axis_name="s")


def _agg_body(h_hbm, src_hbm, dst_hbm, outp, agg_sh, src_v, dring0, dring1,
              rows0, rows1, gsem0, gsem1, ssem0, ssem1, isem0, isem1):
    cid = lax.axis_index("c")
    sid = lax.axis_index("s")
    wid = cid * NS + sid

    zero16 = jnp.zeros((16,), jnp.float32)

    def _zrow(i, _):
        def _zcol(k, __):
            rows0[i, pl.ds(k * 16, 16)] = zero16
            return __
        return lax.fori_loop(0, D // 16, _zcol, 0)
    lax.fori_loop(0, K, _zrow, 0)

    zb = sid * ZPT
    for t in range(ZPT // K):
        pltpu.sync_copy(rows0, agg_sh.at[pl.ds(zb + t * K, K)])
    rem = ZPT % K
    if rem:
        pltpu.sync_copy(rows0.at[pl.ds(0, rem)],
                        agg_sh.at[pl.ds(zb + (ZPT // K) * K, rem)])

    pltpu.sync_copy(src_hbm.at[wid], src_v)

    plsc.subcore_barrier()

    bufs = ((rows0, gsem0, ssem0), (rows1, gsem1, ssem1))
    rings = ((dring0, isem0), (dring1, isem1))
    pltpu.async_copy(dst_hbm.at[wid, pl.ds(0, G)], dring0, isem0)
    pltpu.async_copy(dst_hbm.at[wid, pl.ds(G, G)], dring1, isem1)
    pltpu.async_copy(h_hbm.at[src_v.at[0]], rows0, gsem0)
    pltpu.async_copy(h_hbm.at[src_v.at[1]], rows1, gsem1)

    for g in range(C // G):
        dr, isem = rings[g % 2]
        pltpu.make_async_copy(dst_hbm.at[wid, pl.ds(g * G, G)], dr,
                              isem).wait()

        def _inner(p, carry, g=g, dr=dr):
            j0 = g * G + 2 * p
            r0 = 2 * p
            for b, (rv, gs, ss) in enumerate(bufs):
                j = j0 + b
                pltpu.make_async_copy(h_hbm.at[src_v.at[j]], rv, gs).wait()
                pltpu.async_copy(rv, agg_sh.at[dr.at[r0 + b]], ss, add=True)
            for b, (rv, gs, ss) in enumerate(bufs):
                j = j0 + b
                pltpu.make_async_copy(rv, agg_sh.at[dr.at[r0 + b]],
                                      ss).wait()
                nj = j + 2

                @pl.when(nj < C)
                def _():
                    pltpu.async_copy(h_hbm.at[src_v.at[nj]], rv, gs)
            return carry
        lax.fori_loop(0, G // 2, _inner, 0)

        if g + 2 < C // G:
            pltpu.async_copy(dst_hbm.at[wid, pl.ds((g + 2) * G, G)], dr,
                             isem)

    plsc.subcore_barrier()

    @pl.when(sid < OTILES)
    def _():
        ob = sid * OPT
        pltpu.sync_copy(agg_sh.at[pl.ds(ob, OPT)],
                        outp.at[cid, pl.ds(ob, OPT)])


_agg = pl.kernel(
    _agg_body,
    out_type=[jax.ShapeDtypeStruct((NC, N, D), jnp.float32)],
    mesh=_mesh,
    scratch_types=[
        pltpu.VMEM_SHARED((NPAD, D), jnp.float32),
        pltpu.VMEM((C, K), jnp.int32),
        pltpu.VMEM((G, K), jnp.int32),
        pltpu.VMEM((G, K), jnp.int32),
        pltpu.VMEM((K, D), jnp.float32),
        pltpu.VMEM((K, D), jnp.float32),
        pltpu.SemaphoreType.DMA,
        pltpu.SemaphoreType.DMA,
        pltpu.SemaphoreType.DMA,
        pltpu.SemaphoreType.DMA,
        pltpu.SemaphoreType.DMA,
        pltpu.SemaphoreType.DMA,
    ])


def _deg_body(dst_hbm, degp, deg_sh, dst_v, rows_v, ssem):
    cid = lax.axis_index("c")
    sid = lax.axis_index("s")
    wid = cid * NS + sid

    zero16 = jnp.zeros((16,), jnp.float32)
    one16 = jnp.ones((16,), jnp.float32)

    def _fill(val):
        def _row(i, _):
            def _col(k, __):
                rows_v[i, pl.ds(k * 16, 16)] = val
                return __
            return lax.fori_loop(0, D // 16, _col, 0)
        lax.fori_loop(0, KD, _row, 0)

    _fill(zero16)
    zb = sid * ZPT
    for t in range(ZPT // KD):
        pltpu.sync_copy(rows_v, deg_sh.at[pl.ds(zb + t * KD, KD)])
    rem = ZPT % KD
    if rem:
        pltpu.sync_copy(rows_v.at[pl.ds(0, rem)],
                        deg_sh.at[pl.ds(zb + (ZPT // KD) * KD, rem)])
    _fill(one16)

    pltpu.sync_copy(dst_hbm.at[wid], dst_v)

    plsc.subcore_barrier()

    GB = 8

    def _outer(i, carry):
        for b in range(GB):
            pltpu.async_copy(rows_v, deg_sh.at[dst_v.at[i * GB + b]], ssem,
                             add=True)
        for b in range(GB):
            pltpu.make_async_copy(rows_v,
                                  deg_sh.at[dst_v.at[i * GB + b]],
                                  ssem).wait()
        return carry
    lax.fori_loop(0, CD // GB, _outer, 0)

    plsc.subcore_barrier()

    @pl.when(sid < OTILES)
    def _():
        ob = sid * OPT
        pltpu.sync_copy(deg_sh.at[pl.ds(ob, OPT)],
                        degp.at[cid, pl.ds(ob, OPT)])


_deg = pl.kernel(
    _deg_body,
    out_type=[jax.ShapeDtypeStruct((NC, N, D), jnp.float32)],
    mesh=_mesh,
    scratch_types=[
        pltpu.VMEM_SHARED((NPAD, D), jnp.float32),
        pltpu.VMEM((CD, KD), jnp.int32),
        pltpu.VMEM((KD, D), jnp.float32),
        pltpu.SemaphoreType.DMA,
    ])


BN = 1000
GRID = N // BN


def _norm_body(x_ref, o_ref):
    x = x_ref[...]
    nrm = jnp.sqrt(jnp.sum(x * x, axis=1, keepdims=True))
    o_ref[...] = x / jnp.maximum(nrm, 1e-12)


def _mean_body(h_ref, p0_ref, p1_ref, d0_ref, d1_ref, ws_ref, wn_ref, b_ref,
               o_ref, *, relu):
    deg = d0_ref[:, 0:1] + d1_ref[:, 0:1]
    hn = (p0_ref[...] + p1_ref[...]) / jnp.maximum(deg, 1.0)
    acc = jnp.dot(h_ref[...], ws_ref[...], preferred_element_type=jnp.float32)
    acc = acc + jnp.dot(hn, wn_ref[...], preferred_element_type=jnp.float32)
    acc = acc + b_ref[...]
    o_ref[...] = jnp.maximum(acc, 0.0) if relu else acc


def _gcn_body(h_ref, p0_ref, p1_ref, d0_ref, d1_ref, w_ref, b_ref, o_ref):
    deg = d0_ref[:, 0:1] + d1_ref[:, 0:1]
    h = h_ref[...]
    rst = (h + p0_ref[...] + p1_ref[...]) / (deg + 1.0)
    acc = jnp.dot(rst, w_ref[...], preferred_element_type=jnp.float32)
    o_ref[...] = jnp.maximum(acc + b_ref[...], 0.0)


def _row_spec(d):
    return pl.BlockSpec((BN, d), lambda i: (i, 0))


def _const_spec(r, c):
    return pl.BlockSpec((r, c), lambda i: (0, 0))


_norm_call = pl.pallas_call(
    _norm_body, grid=(GRID,),
    in_specs=[_row_spec(D)], out_specs=_row_spec(D),
    out_shape=jax.ShapeDtypeStruct((N, D), jnp.float32))


def _mean_call(dout, relu):
    return pl.pallas_call(
        functools.partial(_mean_body, relu=relu), grid=(GRID,),
        in_specs=[_row_spec(D), _row_spec(D), _row_spec(D),
                  _row_spec(D), _row_spec(D),
                  _const_spec(D, dout), _const_spec(D, dout),
                  _const_spec(1, dout)],
        out_specs=_row_spec(dout),
        out_shape=jax.ShapeDtypeStruct((N, dout), jnp.float32))


_mean0 = _mean_call(D, True)
_mean3 = _mean_call(64, False)

_gcn = pl.pallas_call(
    _gcn_body, grid=(GRID,),
    in_specs=[_row_spec(D), _row_spec(D), _row_spec(D),
              _row_spec(D), _row_spec(D),
              _const_spec(D, D), _const_spec(1, D)],
    out_specs=_row_spec(D),
    out_shape=jax.ShapeDtypeStruct((N, D), jnp.float32))


def kernel(x, edge_index, Ws0, Wn0, b0, W1, b1, W2, b2, Ws3, Wn3, b3):
    src = edge_index[0].astype(jnp.int32)
    dst = edge_index[1].astype(jnp.int32)
    pad = NW * PAD_T
    pad_src = (jnp.arange(pad, dtype=jnp.int32) * 131) % N
    pad_dst = N + jnp.arange(pad, dtype=jnp.int32) % NDUMMY
    src3 = jnp.concatenate(
        [src.reshape(NW, E // NW), pad_src.reshape(NW, PAD_T)],
        axis=1).reshape(NW, C, K)
    dst2 = jnp.concatenate(
        [dst.reshape(NW, E // NW), pad_dst.reshape(NW, PAD_T)], axis=1)
    dst3 = dst2.reshape(NW, C, K)
    dst3d = dst2.reshape(NW, CD, KD)

    b0r = b0.reshape(1, D)
    b1r = b1.reshape(1, D)
    b2r = b2.reshape(1, D)
    b3r = b3.reshape(1, 64)

    dp, = _deg(dst3d)
    d0, d1 = dp[0], dp[1]
    h = _norm_call(x)
    p, = _agg(h, src3, dst3)
    h = _mean0(h, p[0], p[1], d0, d1, Ws0.T, Wn0.T, b0r)
    p, = _agg(h, src3, dst3)
    h = _gcn(h, p[0], p[1], d0, d1, W1.T, b1r)
    p, = _agg(h, src3, dst3)
    h = _gcn(h, p[0], p[1], d0, d1, W2.T, b2r)
    p, = _agg(h, src3, dst3)
    out = _mean3(h, p[0], p[1], d0, d1, Ws3.T, Wn3.T, b3r)
    return out


# speedup vs baseline: 10.3556x; 1.1696x over previous
import functools

import jax
import jax.numpy as jnp
from jax import lax
from jax.experimental import pallas as pl
from jax.experimental.pallas import tpu as pltpu
from jax.experimental.pallas import tpu_sc as plsc

N = 10000
D = 128
E = 320000
NC = 2
NS = 16
NW = NC * NS
K = 64
EPT = 10240
C = EPT // K
G = 8
NB = 4
KD = 128
CD = EPT // KD
PAD_T = EPT - E // NW
NDUMMY = 16
NPAD = N + NDUMMY
ZPT = NPAD // NS
OPT = 1000
OTILES = N // OPT

_mesh = plsc.VectorSubcoreMesh(core_axis_name="c", subcore_axis_name="s")


def _agg_body(h_hbm, src_hbm, dst_hbm, outp, agg_sh, sring0, sring1, dring0,
              dring1, rows0, rows1, rows2, rows3, gsem0, gsem1, gsem2, gsem3,
              ssem0, ssem1, ssem2, ssem3, isem0, isem1, jsem0, jsem1):
    cid = lax.axis_index("c")
    sid = lax.axis_index("s")
    wid = cid * NS + sid

    zero16 = jnp.zeros((16,), jnp.float32)

    def _zrow(i, _):
        def _zcol(k, __):
            rows0[i, pl.ds(k * 16, 16)] = zero16
            return __
        return lax.fori_loop(0, D // 16, _zcol, 0)
    lax.fori_loop(0, K, _zrow, 0)

    zb = sid * ZPT
    for t in range(ZPT // K):
        pltpu.sync_copy(rows0, agg_sh.at[pl.ds(zb + t * K, K)])
    rem = ZPT % K
    if rem:
        pltpu.sync_copy(rows0.at[pl.ds(0, rem)],
                        agg_sh.at[pl.ds(zb + (ZPT // K) * K, rem)])

    plsc.subcore_barrier()

    bufs = ((rows0, gsem0, ssem0), (rows1, gsem1, ssem1),
            (rows2, gsem2, ssem2), (rows3, gsem3, ssem3))
    rings = ((sring0, jsem0, dring0, isem0), (sring1, jsem1, dring1, isem1))
    pltpu.async_copy(src_hbm.at[wid, pl.ds(0, G)], sring0, jsem0)
    pltpu.async_copy(src_hbm.at[wid, pl.ds(G, G)], sring1, jsem1)
    pltpu.async_copy(dst_hbm.at[wid, pl.ds(0, G)], dring0, isem0)
    pltpu.async_copy(dst_hbm.at[wid, pl.ds(G, G)], dring1, isem1)
    pltpu.make_async_copy(src_hbm.at[wid, pl.ds(0, G)], sring0, jsem0).wait()
    for b, (rv, gs, ss) in enumerate(bufs):
        pltpu.async_copy(h_hbm.at[sring0.at[b]], rv, gs)

    ngroups = C // G
    for g in range(ngroups):
        sr, jsem, dr, isem = rings[g % 2]
        nsr, njsem, ndr, nisem = rings[(g + 1) % 2]
        if g > 0:
            pltpu.make_async_copy(src_hbm.at[wid, pl.ds(g * G, G)], sr,
                                  jsem).wait()
        pltpu.make_async_copy(dst_hbm.at[wid, pl.ds(g * G, G)], dr,
                              isem).wait()

        def _inner(p, carry, g=g, sr=sr, nsr=nsr, dr=dr):
            j0 = g * G + NB * p
            r0 = NB * p
            for b, (rv, gs, ss) in enumerate(bufs):
                j = j0 + b
                pltpu.make_async_copy(h_hbm.at[sr.at[r0 + b]], rv, gs).wait()
                pltpu.async_copy(rv, agg_sh.at[dr.at[r0 + b]], ss, add=True)
            for b, (rv, gs, ss) in enumerate(bufs):
                j = j0 + b
                pltpu.make_async_copy(rv, agg_sh.at[dr.at[r0 + b]],
                                      ss).wait()
                nj = j + NB
                nr = r0 + b + NB
                in_cur = jnp.logical_and(nj < C, nr < G)
                in_nxt = jnp.logical_and(nj < C, nr >= G)

                @pl.when(in_cur)
                def _():
                    pltpu.async_copy(h_hbm.at[sr.at[nr]], rv, gs)

                @pl.when(in_nxt)
                def _():
                    pltpu.async_copy(h_hbm.at[nsr.at[nr - G]], rv, gs)
            return carry
        lax.fori_loop(0, G // NB, _inner, 0)

        if g + 2 < ngroups:
            pltpu.async_copy(src_hbm.at[wid, pl.ds((g + 2) * G, G)], sr,
                             jsem)
            pltpu.async_copy(dst_hbm.at[wid, pl.ds((g + 2) * G, G)], dr,
                             isem)

    plsc.subcore_barrier()

    @pl.when(sid < OTILES)
    def _():
        ob = sid * OPT
        pltpu.sync_copy(agg_sh.at[pl.ds(ob, OPT)],
                        outp.at[cid, pl.ds(ob, OPT)])


_agg = pl.kernel(
    _agg_body,
    out_type=[jax.ShapeDtypeStruct((NC, N, D), jnp.float32)],
    mesh=_mesh,
    scratch_types=[
        pltpu.VMEM_SHARED((NPAD, D), jnp.float32),
        pltpu.VMEM((G, K), jnp.int32),
        pltpu.VMEM((G, K), jnp.int32),
        pltpu.VMEM((G, K), jnp.int32),
        pltpu.VMEM((G, K), jnp.int32),
        pltpu.VMEM((K, D), jnp.float32),
        pltpu.VMEM((K, D), jnp.float32),
        pltpu.VMEM((K, D), jnp.float32),
        pltpu.VMEM((K, D), jnp.float32),
    ] + [pltpu.SemaphoreType.DMA] * 12)


def _deg_body(dst_hbm, degp, deg_sh, dst_v, rows_v, ssem):
    cid = lax.axis_index("c")
    sid = lax.axis_index("s")
    wid = cid * NS + sid

    zero16 = jnp.zeros((16,), jnp.float32)
    one16 = jnp.ones((16,), jnp.float32)

    def _fill(val):
        def _row(i, _):
            def _col(k, __):
                rows_v[i, pl.ds(k * 16, 16)] = val
                return __
            return lax.fori_loop(0, D // 16, _col, 0)
        lax.fori_loop(0, KD, _row, 0)

    _fill(zero16)
    zb = sid * ZPT
    for t in range(ZPT // KD):
        pltpu.sync_copy(rows_v, deg_sh.at[pl.ds(zb + t * KD, KD)])
    rem = ZPT % KD
    if rem:
        pltpu.sync_copy(rows_v.at[pl.ds(0, rem)],
                        deg_sh.at[pl.ds(zb + (ZPT // KD) * KD, rem)])
    _fill(one16)

    pltpu.sync_copy(dst_hbm.at[wid], dst_v)

    plsc.subcore_barrier()

    GB = 8

    def _outer(i, carry):
        for b in range(GB):
            pltpu.async_copy(rows_v, deg_sh.at[dst_v.at[i * GB + b]], ssem,
                             add=True)
        for b in range(GB):
            pltpu.make_async_copy(rows_v,
                                  deg_sh.at[dst_v.at[i * GB + b]],
                                  ssem).wait()
        return carry
    lax.fori_loop(0, CD // GB, _outer, 0)

    plsc.subcore_barrier()

    @pl.when(sid < OTILES)
    def _():
        ob = sid * OPT
        pltpu.sync_copy(deg_sh.at[pl.ds(ob, OPT)],
                        degp.at[cid, pl.ds(ob, OPT)])


_deg = pl.kernel(
    _deg_body,
    out_type=[jax.ShapeDtypeStruct((NC, N, D), jnp.float32)],
    mesh=_mesh,
    scratch_types=[
        pltpu.VMEM_SHARED((NPAD, D), jnp.float32),
        pltpu.VMEM((CD, KD), jnp.int32),
        pltpu.VMEM((KD, D), jnp.float32),
        pltpu.SemaphoreType.DMA,
    ])


BN = 1000
GRID = N // BN


def _norm_body(x_ref, o_ref):
    x = x_ref[...]
    nrm = jnp.sqrt(jnp.sum(x * x, axis=1, keepdims=True))
    o_ref[...] = x / jnp.maximum(nrm, 1e-12)


def _mean_body(h_ref, p0_ref, p1_ref, d0_ref, d1_ref, ws_ref, wn_ref, b_ref,
               o_ref, *, relu):
    deg = d0_ref[:, 0:1] + d1_ref[:, 0:1]
    hn = (p0_ref[...] + p1_ref[...]) / jnp.maximum(deg, 1.0)
    acc = jnp.dot(h_ref[...], ws_ref[...], preferred_element_type=jnp.float32)
    acc = acc + jnp.dot(hn, wn_ref[...], preferred_element_type=jnp.float32)
    acc = acc + b_ref[...]
    o_ref[...] = jnp.maximum(acc, 0.0) if relu else acc


def _gcn_body(h_ref, p0_ref, p1_ref, d0_ref, d1_ref, w_ref, b_ref, o_ref):
    deg = d0_ref[:, 0:1] + d1_ref[:, 0:1]
    h = h_ref[...]
    rst = (h + p0_ref[...] + p1_ref[...]) / (deg + 1.0)
    acc = jnp.dot(rst, w_ref[...], preferred_element_type=jnp.float32)
    o_ref[...] = jnp.maximum(acc + b_ref[...], 0.0)


def _row_spec(d):
    return pl.BlockSpec((BN, d), lambda i: (i, 0))


def _const_spec(r, c):
    return pl.BlockSpec((r, c), lambda i: (0, 0))


_norm_call = pl.pallas_call(
    _norm_body, grid=(GRID,),
    in_specs=[_row_spec(D)], out_specs=_row_spec(D),
    out_shape=jax.ShapeDtypeStruct((N, D), jnp.float32))


def _mean_call(dout, relu):
    return pl.pallas_call(
        functools.partial(_mean_body, relu=relu), grid=(GRID,),
        in_specs=[_row_spec(D), _row_spec(D), _row_spec(D),
                  _row_spec(D), _row_spec(D),
                  _const_spec(D, dout), _const_spec(D, dout),
                  _const_spec(1, dout)],
        out_specs=_row_spec(dout),
        out_shape=jax.ShapeDtypeStruct((N, dout), jnp.float32))


_mean0 = _mean_call(D, True)
_mean3 = _mean_call(64, False)

_gcn = pl.pallas_call(
    _gcn_body, grid=(GRID,),
    in_specs=[_row_spec(D), _row_spec(D), _row_spec(D),
              _row_spec(D), _row_spec(D),
              _const_spec(D, D), _const_spec(1, D)],
    out_specs=_row_spec(D),
    out_shape=jax.ShapeDtypeStruct((N, D), jnp.float32))


def kernel(x, edge_index, Ws0, Wn0, b0, W1, b1, W2, b2, Ws3, Wn3, b3):
    src = edge_index[0].astype(jnp.int32)
    dst = edge_index[1].astype(jnp.int32)
    pad = NW * PAD_T
    pad_src = (jnp.arange(pad, dtype=jnp.int32) * 131) % N
    pad_dst = N + jnp.arange(pad, dtype=jnp.int32) % NDUMMY
    src3 = jnp.concatenate(
        [src.reshape(NW, E // NW), pad_src.reshape(NW, PAD_T)],
        axis=1).reshape(NW, C, K)
    dst2 = jnp.concatenate(
        [dst.reshape(NW, E // NW), pad_dst.reshape(NW, PAD_T)], axis=1)
    dst3 = dst2.reshape(NW, C, K)
    dst3d = dst2.reshape(NW, CD, KD)

    b0r = b0.reshape(1, D)
    b1r = b1.reshape(1, D)
    b2r = b2.reshape(1, D)
    b3r = b3.reshape(1, 64)

    dp, = _deg(dst3d)
    d0, d1 = dp[0], dp[1]
    h = _norm_call(x)
    p, = _agg(h, src3, dst3)
    h = _mean0(h, p[0], p[1], d0, d1, Ws0.T, Wn0.T, b0r)
    p, = _agg(h, src3, dst3)
    h = _gcn(h, p[0], p[1], d0, d1, W1.T, b1r)
    p, = _agg(h, src3, dst3)
    h = _gcn(h, p[0], p[1], d0, d1, W2.T, b2r)
    p, = _agg(h, src3, dst3)
    out = _mean3(h, p[0], p[1], d0, d1, Ws3.T, Wn3.T, b3r)
    return out
